# Initial kernel scaffold; baseline (speedup 1.0000x reference)
import jax
import jax.numpy as jnp
from jax.experimental import pallas as pl


def kernel(input0, input1):
    raise NotImplementedError("write your pallas kernel here")


# speedup vs baseline: 1.3326x; 1.3326x over previous
import functools

import jax
import jax.numpy as jnp
from jax.experimental import pallas as pl
from jax.experimental.pallas import tpu as pltpu

_N = 8192
_BM = 512


def _chamfer_body(a_ref, bt_ref, out_ref, d1_acc, s0_acc):
    i = pl.program_id(0)
    ni = pl.num_programs(0)

    a = a_ref[...]
    bt = bt_ref[...]
    n0 = jnp.sum(a * a, axis=1, keepdims=True)
    n1 = jnp.sum(bt * bt, axis=0, keepdims=True)
    prod = jnp.dot(a, bt, preferred_element_type=jnp.float32)
    d2 = n0 + n1 - 2.0 * prod

    row_min = jnp.min(d2, axis=1)
    col_min = jnp.min(d2, axis=0, keepdims=True)

    @pl.when(i == 0)
    def _init():
        d1_acc[...] = col_min
        s0_acc[0, 0] = 0.0

    @pl.when(i != 0)
    def _accum():
        d1_acc[...] = jnp.minimum(d1_acc[...], col_min)

    s0_acc[0, 0] += jnp.sum(jnp.maximum(row_min, 0.0))

    @pl.when(i == ni - 1)
    def _finish():
        d1_sum = jnp.sum(jnp.maximum(d1_acc[...], 0.0))
        loss = (s0_acc[0, 0] + d1_sum) / float(_N)
        out_ref[...] = jnp.broadcast_to(loss, (1, 1))


def _chamfer(pc0, pc1t):
    ni = _N // _BM
    out = pl.pallas_call(
        _chamfer_body,
        grid=(ni,),
        in_specs=[
            pl.BlockSpec((_BM, 3), lambda i: (i, 0)),
            pl.BlockSpec((3, _N), lambda i: (0, 0)),
        ],
        out_specs=pl.BlockSpec((1, 1), lambda i: (0, 0)),
        out_shape=jax.ShapeDtypeStruct((1, 1), jnp.float32),
        scratch_shapes=[
            pltpu.VMEM((1, _N), jnp.float32),
            pltpu.SMEM((1, 1), jnp.float32),
        ],
    )(pc0, pc1t)
    return out[0, 0]


@jax.jit
def kernel(input0, input1):
    pc1t = input1.T
    return _chamfer(input0, pc1t)


# speedup vs baseline: 1.7289x; 1.2974x over previous
import functools

import jax
import jax.numpy as jnp
from jax.experimental import pallas as pl
from jax.experimental.pallas import tpu as pltpu

_N = 8192
_BM = 512


def _chamfer_body(a_ref, bt_ref, out_ref, d1_acc, s0_acc):
    i = pl.program_id(0)
    ni = pl.num_programs(0)

    a = a_ref[...]
    bt = bt_ref[...]
    d2 = jnp.dot(a, bt, preferred_element_type=jnp.float32)

    row_min = jnp.min(d2, axis=1)
    col_min = jnp.min(d2, axis=0, keepdims=True)

    @pl.when(i == 0)
    def _init():
        d1_acc[...] = col_min
        s0_acc[0, 0] = 0.0

    @pl.when(i != 0)
    def _accum():
        d1_acc[...] = jnp.minimum(d1_acc[...], col_min)

    s0_acc[0, 0] += jnp.sum(jnp.maximum(row_min, 0.0))

    @pl.when(i == ni - 1)
    def _finish():
        d1_sum = jnp.sum(jnp.maximum(d1_acc[...], 0.0))
        loss = (s0_acc[0, 0] + d1_sum) / float(_N)
        out_ref[...] = jnp.broadcast_to(loss, (1, 1))


def _chamfer(pc0, pc1t):
    ni = _N // _BM
    out = pl.pallas_call(
        _chamfer_body,
        grid=(ni,),
        in_specs=[
            pl.BlockSpec((_BM, 5), lambda i: (i, 0)),
            pl.BlockSpec((5, _N), lambda i: (0, 0)),
        ],
        out_specs=pl.BlockSpec((1, 1), lambda i: (0, 0)),
        out_shape=jax.ShapeDtypeStruct((1, 1), jnp.float32),
        scratch_shapes=[
            pltpu.VMEM((1, _N), jnp.float32),
            pltpu.SMEM((1, 1), jnp.float32),
        ],
    )(pc0, pc1t)
    return out[0, 0]


@jax.jit
def kernel(input0, input1):
    n0 = jnp.sum(input0 * input0, axis=1, keepdims=True)
    n1 = jnp.sum(input1 * input1, axis=1)[None, :]
    ones_col = jnp.ones((_N, 1), jnp.float32)
    a5 = jnp.concatenate([-2.0 * input0, ones_col, n0], axis=1)
    b5t = jnp.concatenate([input1.T, n1, ones_col.T], axis=0)
    return _chamfer(a5, b5t)
